# baseline (speedup 1.0000x reference)
import functools

import jax
import jax.numpy as jnp
from jax import lax
from jax.experimental import pallas as pl
from jax.experimental.pallas import tpu as pltpu
from jax.experimental.pallas import tpu_sc as plsc

_D = 64
_S = 50
_SG = 56
_NB = 8
_NC = 2
_NS = 16
_NW = _NC * _NS


_SP = 128


@functools.partial(jax.jit, static_argnums=(2,))
def _sc_gather(table, idx_padded, s):
    b = idx_padded.shape[0]
    b_per_w = b // _NW
    nchunks = b_per_w // _NB
    mesh = plsc.VectorSubcoreMesh(core_axis_name="c", subcore_axis_name="s")

    @functools.partial(
        pl.kernel,
        out_type=jax.ShapeDtypeStruct((b, s, _D), jnp.float32),
        mesh=mesh,
        scratch_types=[
            pltpu.VMEM((_NB, _SP), jnp.int32),
            pltpu.VMEM((_NB, _SG, _D), jnp.float32),
            pltpu.SemaphoreType.DMA,
        ],
        compiler_params=pltpu.CompilerParams(use_tc_tiling_on_sc=False),
    )
    def k(table_hbm, idx_hbm, out_hbm, idx_v, rows_v, gsem):
        wid = lax.axis_index("s") * _NC + lax.axis_index("c")
        base = wid * b_per_w

        def body(c, carry):
            b0 = base + c * _NB
            pltpu.sync_copy(idx_hbm.at[pl.ds(b0, _NB)], idx_v)
            copies = [
                pltpu.async_copy(
                    table_hbm.at[idx_v.at[j, pl.ds(0, _SG)]],
                    rows_v.at[j],
                    gsem,
                )
                for j in range(_NB)
            ]
            for cp in copies:
                cp.wait()
            outs = [
                pltpu.async_copy(
                    rows_v.at[j, pl.ds(0, _S)],
                    out_hbm.at[b0 + j],
                    gsem,
                )
                for j in range(_NB)
            ]
            for cp in outs:
                cp.wait()
            return carry

        lax.fori_loop(0, nchunks, body, 0)

    return k(table, idx_padded)


def kernel(input_tensor, indices):
    b, s = indices.shape
    idx_padded = jnp.pad(indices.astype(jnp.int32), ((0, 0), (0, _SP - s)))
    return _sc_gather(input_tensor, idx_padded, s)


# speedup vs baseline: 1.6817x; 1.6817x over previous
import functools

import jax
import jax.numpy as jnp
from jax import lax
from jax.experimental import pallas as pl
from jax.experimental.pallas import tpu as pltpu
from jax.experimental.pallas import tpu_sc as plsc

_D = 64
_GRP = 128
_K = 5
_CHUNK = _GRP * _K
_NC = 2
_NS = 16
_NW = _NC * _NS


@jax.jit
def _sc_gather(table, idx_groups):
    num_groups = idx_groups.shape[0]
    n = num_groups * _GRP
    rows_per_w = n // _NW
    nchunks = rows_per_w // _CHUNK
    groups_per_chunk = _K
    mesh = plsc.VectorSubcoreMesh(core_axis_name="c", subcore_axis_name="s")

    @functools.partial(
        pl.kernel,
        out_type=jax.ShapeDtypeStruct((n, _D), jnp.float32),
        mesh=mesh,
        scratch_types=[
            pltpu.VMEM((_K, _GRP), jnp.int32),
            pltpu.VMEM((_CHUNK, _D), jnp.float32),
            pltpu.SemaphoreType.DMA,
        ],
        compiler_params=pltpu.CompilerParams(use_tc_tiling_on_sc=False),
    )
    def k(table_hbm, idx_hbm, out_hbm, idx_v, rows_v, gsem):
        wid = lax.axis_index("s") * _NC + lax.axis_index("c")
        gbase = wid * (rows_per_w // _GRP)

        def body(c, carry):
            g0 = gbase + c * groups_per_chunk
            pltpu.sync_copy(idx_hbm.at[pl.ds(g0, groups_per_chunk)], idx_v)
            copies = [
                pltpu.async_copy(
                    table_hbm.at[idx_v.at[j]],
                    rows_v.at[pl.ds(j * _GRP, _GRP)],
                    gsem,
                )
                for j in range(_K)
            ]
            for cp in copies:
                cp.wait()
            pltpu.sync_copy(rows_v, out_hbm.at[pl.ds(g0 * _GRP, _CHUNK)])
            return carry

        lax.fori_loop(0, nchunks, body, 0)

    return k(table, idx_groups)


def kernel(input_tensor, indices):
    b, s = indices.shape
    n = b * s
    idx_groups = jnp.minimum(
        indices.astype(jnp.int32).reshape(n // _GRP, _GRP),
        input_tensor.shape[0] - 1,
    )
    out = _sc_gather(input_tensor, idx_groups)
    return out.reshape(b, s, _D)
